# baseline (device time: 63707 ns/iter reference)
import jax
import jax.numpy as jnp
from jax import lax
from jax.experimental import pallas as pl
from jax.experimental.pallas import tpu as pltpu

N_DEV = 8
B, SQ, SKV, HQ, DH = 2, 128, 1024, 4, 64
SKV_SH = SKV // N_DEV
D_MODEL = HQ * DH
EVEN_ORIGINS = (0, 2, 4, 6)


def kernel(x, Wq, K_ext, V_ext, Wo):
    def body(x_ref, wq_ref, k_ref, v_ref, wo_ref, out_ref,
             kvg_ref, send_sems, recv_sems):
        my = lax.axis_index("i")
        left = (my + N_DEV - 1) % N_DEV
        right = (my + 1) % N_DEV

        barrier_sem = pltpu.get_barrier_semaphore()
        for nbr in (left, right):
            pl.semaphore_signal(
                barrier_sem, inc=1,
                device_id=(nbr,), device_id_type=pl.DeviceIdType.MESH,
            )
        pl.semaphore_wait(barrier_sem, 2)

        kvg_ref[my, 0] = k_ref[...].astype(jnp.bfloat16)
        kvg_ref[my, 1] = v_ref[...].astype(jnp.bfloat16)

        for h in range(N_DEV - 1):
            src_o = (my + N_DEV - h) % N_DEV
            rdma = pltpu.make_async_remote_copy(
                src_ref=kvg_ref.at[src_o],
                dst_ref=kvg_ref.at[src_o],
                send_sem=send_sems.at[h],
                recv_sem=recv_sems.at[h],
                device_id=(right,),
                device_id_type=pl.DeviceIdType.MESH,
            )
            rdma.start()
            rdma.wait()

        rows = lax.broadcasted_iota(jnp.int32, (SQ, 4 * SKV_SH), 0)
        cols = lax.broadcasted_iota(jnp.int32, (SQ, 4 * SKV_SH), 1)
        valid = ((cols % SKV_SH) // 64) == (rows // 64)

        wq = wq_ref[...].astype(jnp.bfloat16)
        wo = wo_ref[...].astype(jnp.bfloat16)
        for b in range(B):
            q_b = lax.dot_general(
                x_ref[b].astype(jnp.bfloat16), wq,
                (((1,), (0,)), ((), ())),
                preferred_element_type=jnp.float32,
            ).astype(jnp.bfloat16)
            head_ctx = []
            for hh in range(HQ):
                q = q_b[:, hh * DH:(hh + 1) * DH]
                blocks = []
                for o in EVEN_ORIGINS:
                    kk = kvg_ref[o, 0, b].reshape(SKV_SH, D_MODEL)
                    kk = kk[:, hh * DH:(hh + 1) * DH]
                    blocks.append(lax.dot_general(
                        q, kk, (((1,), (1,)), ((), ())),
                        preferred_element_type=jnp.float32))
                scores = jnp.concatenate(blocks, axis=1) * 0.125
                scores = jnp.where(valid, scores, -1e9)
                m = jnp.max(scores, axis=1, keepdims=True)
                w = jnp.exp(scores - m)
                w = w / jnp.sum(w, axis=1, keepdims=True)
                w = w.astype(jnp.bfloat16)
                ctx = jnp.zeros((SQ, DH), jnp.float32)
                for oi, o in enumerate(EVEN_ORIGINS):
                    vv = kvg_ref[o, 1, b].reshape(SKV_SH, D_MODEL)
                    vv = vv[:, hh * DH:(hh + 1) * DH]
                    ctx += lax.dot_general(
                        w[:, oi * SKV_SH:(oi + 1) * SKV_SH], vv,
                        (((1,), (0,)), ((), ())),
                        preferred_element_type=jnp.float32)
                head_ctx.append(ctx.astype(jnp.bfloat16))
            ctx_b = jnp.concatenate(head_ctx, axis=1)
            out_ref[b] = lax.dot_general(
                ctx_b, wo, (((1,), (0,)), ((), ())),
                preferred_element_type=jnp.float32)

    return pl.pallas_call(
        body,
        out_shape=jax.ShapeDtypeStruct((B, SQ, 512), jnp.float32),
        in_specs=[pl.BlockSpec(memory_space=pltpu.VMEM)] * 5,
        out_specs=pl.BlockSpec(memory_space=pltpu.VMEM),
        scratch_shapes=[
            pltpu.VMEM((N_DEV, 2, B, SKV_SH, HQ, DH), jnp.bfloat16),
            pltpu.SemaphoreType.DMA((N_DEV - 1,)),
            pltpu.SemaphoreType.DMA((N_DEV - 1,)),
        ],
        compiler_params=pltpu.CompilerParams(collective_id=0),
    )(x, Wq, K_ext, V_ext, Wo)


# device time: 25233 ns/iter; 2.5247x vs baseline; 2.5247x over previous
import jax
import jax.numpy as jnp
from jax import lax
from jax.experimental import pallas as pl
from jax.experimental.pallas import tpu as pltpu

N_DEV = 8
N_ROUNDS = 3
B, SQ, SKV, HQ, DH = 2, 128, 1024, 4, 64
SKV_SH = SKV // N_DEV
D_MODEL = HQ * DH
NEG = -1e9


def kernel(x, Wq, K_ext, V_ext, Wo):
    def body(x_ref, wq_ref, k_ref, v_ref, wo_ref, out_ref,
             ctx_buf, ctx_rcv, st_buf, st_rcv,
             ctx_ssem, ctx_rsem, st_ssem, st_rsem):
        my = lax.axis_index("i")

        barrier_sem = pltpu.get_barrier_semaphore()
        for r in range(N_ROUNDS):
            pl.semaphore_signal(
                barrier_sem, inc=1,
                device_id=(my ^ (1 << r),),
                device_id_type=pl.DeviceIdType.MESH,
            )
        pl.semaphore_wait(barrier_sem, N_ROUNDS)

        rows = lax.broadcasted_iota(jnp.int32, (SQ, SKV_SH), 0)
        cols = lax.broadcasted_iota(jnp.int32, (SQ, SKV_SH), 1)
        qb = rows // 64
        kb = 2 * my + cols // 64
        valid = (kb == qb) | (kb % 4 == qb)

        wq = wq_ref[...].astype(jnp.bfloat16)
        for b in range(B):
            q_b = lax.dot_general(
                x_ref[b].astype(jnp.bfloat16), wq,
                (((1,), (0,)), ((), ())),
                preferred_element_type=jnp.float32,
            ).astype(jnp.bfloat16)
            kv_k = k_ref[b].astype(jnp.bfloat16).reshape(SKV_SH, D_MODEL)
            kv_v = v_ref[b].astype(jnp.bfloat16).reshape(SKV_SH, D_MODEL)
            for hh in range(HQ):
                q = q_b[:, hh * DH:(hh + 1) * DH]
                kk = kv_k[:, hh * DH:(hh + 1) * DH]
                vv = kv_v[:, hh * DH:(hh + 1) * DH]
                s = lax.dot_general(
                    q, kk, (((1,), (1,)), ((), ())),
                    preferred_element_type=jnp.float32) * 0.125
                s = jnp.where(valid, s, NEG)
                m = jnp.max(s, axis=1)
                w = jnp.exp(s - m[:, None])
                ssum = jnp.sum(w, axis=1)
                ctx = lax.dot_general(
                    w.astype(jnp.bfloat16), vv,
                    (((1,), (0,)), ((), ())),
                    preferred_element_type=jnp.float32)
                ctx_buf[0, b, hh] = ctx.astype(jnp.bfloat16)
                st_buf[0, 0, b, hh] = m
                st_buf[0, 1, b, hh] = ssum

        c_new = None
        s_new = None
        for r in range(N_ROUNDS):
            partner = my ^ (1 << r)
            ctx_rdma = pltpu.make_async_remote_copy(
                src_ref=ctx_buf.at[r], dst_ref=ctx_rcv.at[r],
                send_sem=ctx_ssem.at[r], recv_sem=ctx_rsem.at[r],
                device_id=(partner,), device_id_type=pl.DeviceIdType.MESH,
            )
            st_rdma = pltpu.make_async_remote_copy(
                src_ref=st_buf.at[r], dst_ref=st_rcv.at[r],
                send_sem=st_ssem.at[r], recv_sem=st_rsem.at[r],
                device_id=(partner,), device_id_type=pl.DeviceIdType.MESH,
            )
            ctx_rdma.start()
            st_rdma.start()
            ctx_rdma.wait()
            st_rdma.wait()

            m1 = st_buf[r, 0]
            s1 = st_buf[r, 1]
            m2 = st_rcv[r, 0]
            s2 = st_rcv[r, 1]
            mm = jnp.maximum(m1, m2)
            a1 = jnp.exp(m1 - mm)
            a2 = jnp.exp(m2 - mm)
            s_new = a1 * s1 + a2 * s2
            c_new = (ctx_buf[r].astype(jnp.float32) * a1[..., None]
                     + ctx_rcv[r].astype(jnp.float32) * a2[..., None])
            if r + 1 < N_ROUNDS:
                ctx_buf[r + 1] = c_new.astype(jnp.bfloat16)
                st_buf[r + 1, 0] = mm
                st_buf[r + 1, 1] = s_new

        ctx_n = c_new / s_new[..., None]
        wo = wo_ref[...].astype(jnp.bfloat16)
        for b in range(B):
            ctx_b = jnp.concatenate(
                [ctx_n[b, hh] for hh in range(HQ)], axis=1)
            out_ref[b] = lax.dot_general(
                ctx_b.astype(jnp.bfloat16), wo, (((1,), (0,)), ((), ())),
                preferred_element_type=jnp.float32)

    return pl.pallas_call(
        body,
        out_shape=jax.ShapeDtypeStruct((B, SQ, 512), jnp.float32),
        in_specs=[pl.BlockSpec(memory_space=pltpu.VMEM)] * 5,
        out_specs=pl.BlockSpec(memory_space=pltpu.VMEM),
        scratch_shapes=[
            pltpu.VMEM((N_ROUNDS, B, HQ, SQ, DH), jnp.bfloat16),
            pltpu.VMEM((N_ROUNDS, B, HQ, SQ, DH), jnp.bfloat16),
            pltpu.VMEM((N_ROUNDS, 2, B, HQ, SQ), jnp.float32),
            pltpu.VMEM((N_ROUNDS, 2, B, HQ, SQ), jnp.float32),
            pltpu.SemaphoreType.DMA((N_ROUNDS,)),
            pltpu.SemaphoreType.DMA((N_ROUNDS,)),
            pltpu.SemaphoreType.DMA((N_ROUNDS,)),
            pltpu.SemaphoreType.DMA((N_ROUNDS,)),
        ],
        compiler_params=pltpu.CompilerParams(collective_id=0),
    )(x, Wq, K_ext, V_ext, Wo)
